# baseline (speedup 1.0000x reference)
import jax
import jax.numpy as jnp
import numpy as np
from jax.experimental import pallas as pl
from jax.experimental.pallas import tpu as pltpu

_BINS = 10
_N, _C = 131072, 80
_BLOCK_ROWS = 16384
_NUM_BLOCKS = _N // _BLOCK_ROWS
_CHUNK = 16
_NUM_CHUNKS = _BLOCK_ROWS // _CHUNK


def _fold8(x):
    r = x.shape[0]
    while r > 8:
        half = r // 2
        x = x[:half, :] + x[half:, :]
        r = half
    return x

_EDGES_F32 = (np.arange(_BINS + 1, dtype=np.float32) / np.float32(_BINS))
_INNER_EDGES = [float(_EDGES_F32[i]) for i in range(1, _BINS)]


def _ghm_body(pred_ref, targ_ref, out_ref, cnt_ref, sum_ref):
    i = pl.program_id(0)

    @pl.when(i == 0)
    def _init():
        cnt_ref[...] = jnp.zeros_like(cnt_ref)
        sum_ref[...] = jnp.zeros_like(sum_ref)

    zero_acc = jnp.zeros((_CHUNK, _C), jnp.float32)

    def chunk_step(k, carry):
        cnts, sums, tot = carry
        p = pred_ref[pl.ds(k * _CHUNK, _CHUNK), :]
        t = targ_ref[pl.ds(k * _CHUNK, _CHUNK), :]
        e = jnp.exp(-jnp.abs(p))
        denom = 1.0 + e
        s = jnp.where(p >= 0.0, 1.0, e) / denom
        g = jnp.abs(s - t)
        bce = jnp.maximum(p, 0.0) - p * t + jnp.log(denom)
        tot = tot + bce
        new_cnts = []
        new_sums = []
        for b in range(9):
            m = jnp.where(g >= _INNER_EDGES[b], 1.0, 0.0)
            new_cnts.append(cnts[b] + m)
            new_sums.append(sums[b] + bce * m)
        return tuple(new_cnts), tuple(new_sums), tot

    init = (
        tuple(zero_acc for _ in range(9)),
        tuple(zero_acc for _ in range(9)),
        zero_acc,
    )
    cnts, sums, tot = jax.lax.fori_loop(0, _NUM_CHUNKS, chunk_step, init)

    sum_ref[0:8, :] += _fold8(tot)
    for b in range(9):
        cnt_ref[8 * b : 8 * b + 8, :] += _fold8(cnts[b])
        sum_ref[8 * (b + 1) : 8 * (b + 1) + 8, :] += _fold8(sums[b])

    @pl.when(i == _NUM_BLOCKS - 1)
    def _finish():
        total = jnp.float32(_N * _C)
        lvl_cnt = [total] + [
            jnp.sum(cnt_ref[8 * b : 8 * b + 8, :]) for b in range(9)
        ]
        lvl_sum = [jnp.sum(sum_ref[8 * b : 8 * b + 8, :]) for b in range(10)]
        n = jnp.float32(0.0)
        acc = jnp.float32(0.0)
        for b in range(_BINS):
            hi_c = jnp.float32(0.0) if b == _BINS - 1 else lvl_cnt[b + 1]
            hi_s = jnp.float32(0.0) if b == _BINS - 1 else lvl_sum[b + 1]
            c = lvl_cnt[b] - hi_c
            sb = lvl_sum[b] - hi_s
            nonempty = c > 0.0
            n = n + jnp.where(nonempty, 1.0, 0.0)
            acc = acc + jnp.where(nonempty, sb / jnp.maximum(c, 1.0), 0.0)
        out_ref[0] = jnp.where(n > 0.0, acc / jnp.maximum(n, 1.0), acc)


@jax.jit
def kernel(pred, target):
    out = pl.pallas_call(
        _ghm_body,
        grid=(_NUM_BLOCKS,),
        in_specs=[
            pl.BlockSpec((_BLOCK_ROWS, _C), lambda i: (i, 0)),
            pl.BlockSpec((_BLOCK_ROWS, _C), lambda i: (i, 0)),
        ],
        out_specs=pl.BlockSpec(memory_space=pltpu.SMEM),
        out_shape=jax.ShapeDtypeStruct((1,), jnp.float32),
        scratch_shapes=[
            pltpu.VMEM((72, _C), jnp.float32),
            pltpu.VMEM((80, _C), jnp.float32),
        ],
    )(pred, target)
    return jnp.reshape(out, ())


# speedup vs baseline: 1.7822x; 1.7822x over previous
import jax
import jax.numpy as jnp
import numpy as np
from jax.experimental import pallas as pl
from jax.experimental.pallas import tpu as pltpu

_BINS = 10
_N, _C = 131072, 80
_BLOCK_ROWS = 16384
_NUM_BLOCKS = _N // _BLOCK_ROWS
_CHUNK = 128
_NUM_CHUNKS = _BLOCK_ROWS // _CHUNK


def _fold8(x):
    r = x.shape[0]
    while r > 8:
        half = r // 2
        x = x[:half, :] + x[half:, :]
        r = half
    return x

_EDGES_F32 = (np.arange(_BINS + 1, dtype=np.float32) / np.float32(_BINS))
_INNER_EDGES = [float(_EDGES_F32[i]) for i in range(1, _BINS)]


def _ghm_body(pred_ref, targ_ref, out_ref, cnt_ref, sum_ref):
    i = pl.program_id(0)

    @pl.when(i == 0)
    def _init():
        cnt_ref[...] = jnp.zeros_like(cnt_ref)
        sum_ref[...] = jnp.zeros_like(sum_ref)

    zero_vreg = jnp.zeros((8, _C), jnp.float32)

    def chunk_step(k, carry):
        cnts, sums, tot = carry
        p = pred_ref[pl.ds(k * _CHUNK, _CHUNK), :]
        t = targ_ref[pl.ds(k * _CHUNK, _CHUNK), :]
        e = jnp.exp(-jnp.abs(p))
        denom = 1.0 + e
        s = jnp.where(p >= 0.0, 1.0, e) / denom
        g = jnp.abs(s - t)
        bce = jnp.maximum(p, 0.0) - p * t + jnp.log(denom)
        tot = tot + _fold8(bce)
        new_cnts = []
        new_sums = []
        for b in range(9):
            hit = g >= _INNER_EDGES[b]
            new_cnts.append(cnts[b] + _fold8(jnp.where(hit, 1.0, 0.0)))
            new_sums.append(sums[b] + _fold8(jnp.where(hit, bce, 0.0)))
        return tuple(new_cnts), tuple(new_sums), tot

    init = (
        tuple(zero_vreg for _ in range(9)),
        tuple(zero_vreg for _ in range(9)),
        zero_vreg,
    )
    cnts, sums, tot = jax.lax.fori_loop(0, _NUM_CHUNKS, chunk_step, init)

    sum_ref[0:8, :] += tot
    for b in range(9):
        cnt_ref[8 * b : 8 * b + 8, :] += cnts[b]
        sum_ref[8 * (b + 1) : 8 * (b + 1) + 8, :] += sums[b]

    @pl.when(i == _NUM_BLOCKS - 1)
    def _finish():
        total = jnp.float32(_N * _C)
        lvl_cnt = [total] + [
            jnp.sum(cnt_ref[8 * b : 8 * b + 8, :]) for b in range(9)
        ]
        lvl_sum = [jnp.sum(sum_ref[8 * b : 8 * b + 8, :]) for b in range(10)]
        n = jnp.float32(0.0)
        acc = jnp.float32(0.0)
        for b in range(_BINS):
            hi_c = jnp.float32(0.0) if b == _BINS - 1 else lvl_cnt[b + 1]
            hi_s = jnp.float32(0.0) if b == _BINS - 1 else lvl_sum[b + 1]
            c = lvl_cnt[b] - hi_c
            sb = lvl_sum[b] - hi_s
            nonempty = c > 0.0
            n = n + jnp.where(nonempty, 1.0, 0.0)
            acc = acc + jnp.where(nonempty, sb / jnp.maximum(c, 1.0), 0.0)
        out_ref[0] = jnp.where(n > 0.0, acc / jnp.maximum(n, 1.0), acc)


@jax.jit
def kernel(pred, target):
    out = pl.pallas_call(
        _ghm_body,
        grid=(_NUM_BLOCKS,),
        in_specs=[
            pl.BlockSpec((_BLOCK_ROWS, _C), lambda i: (i, 0)),
            pl.BlockSpec((_BLOCK_ROWS, _C), lambda i: (i, 0)),
        ],
        out_specs=pl.BlockSpec(memory_space=pltpu.SMEM),
        out_shape=jax.ShapeDtypeStruct((1,), jnp.float32),
        scratch_shapes=[
            pltpu.VMEM((72, _C), jnp.float32),
            pltpu.VMEM((80, _C), jnp.float32),
        ],
    )(pred, target)
    return jnp.reshape(out, ())
